# baseline (speedup 1.0000x reference)
import functools

import jax
import jax.numpy as jnp
from jax import lax
from jax.experimental import pallas as pl
from jax.experimental.pallas import tpu as pltpu
from jax.experimental.pallas import tpu_sc as plsc

N = 10000
E = 320000
D = 128
G = 64

NC = 2
NS = 16
NW = NC * NS
K = 112
NCHUNK = 92
EPW = NCHUNK * K
EPAD = NW * EPW
NPAD = 10240
RPT = NPAD // NS
DUMMY = NPAD - 8


def _sc_aggregate(h, src, dst, zeros):
    mesh = plsc.VectorSubcoreMesh(core_axis_name="c", subcore_axis_name="s")

    @functools.partial(
        pl.kernel,
        out_type=jax.ShapeDtypeStruct((NC, NPAD, D), jnp.float32),
        mesh=mesh,
        scratch_types=[
            pltpu.VMEM((K,), jnp.int32),
            pltpu.VMEM((K,), jnp.int32),
            pltpu.VMEM((K, D), jnp.float32),
            pltpu.VMEM_SHARED((NPAD, D), jnp.float32),
            pltpu.SemaphoreType.DMA,
        ],
    )
    def agg(h_hbm, src_hbm, dst_hbm, zero_hbm, out_hbm,
            src_v, dst_v, rows_v, acc_sh, sem):
        c = lax.axis_index("c")
        s = lax.axis_index("s")
        wid = s * NC + c
        r0 = s * RPT
        pltpu.sync_copy(zero_hbm.at[pl.ds(r0, RPT)], acc_sh.at[pl.ds(r0, RPT)])
        plsc.subcore_barrier()

        base0 = wid * EPW

        @pl.loop(0, NCHUNK)
        def chunk(i):
            base = base0 + i * K
            pltpu.sync_copy(src_hbm.at[pl.ds(base, K)], src_v)
            pltpu.sync_copy(dst_hbm.at[pl.ds(base, K)], dst_v)
            pltpu.async_copy(h_hbm.at[src_v], rows_v, sem).wait()
            pltpu.sync_copy(rows_v, acc_sh.at[dst_v], add=True)

        plsc.subcore_barrier()
        pltpu.sync_copy(acc_sh.at[pl.ds(r0, RPT)],
                        out_hbm.at[c, pl.ds(r0, RPT)])

    return agg(h, src, dst, zeros)


def _tc_layer_body(h_ref, a_ref, batch_ref, W1_ref, b1_ref, g_ref, be_ref,
                   W2_ref, b2_ref, hout_ref, pooled_ref):
    z = h_ref[...] + a_ref[0, :N] + a_ref[1, :N]
    t = jnp.dot(z, W1_ref[...], preferred_element_type=jnp.float32)
    t = t + b1_ref[...]
    m = jnp.mean(t, axis=0, keepdims=True)
    v = jnp.mean((t - m) * (t - m), axis=0, keepdims=True)
    t = (t - m) / jnp.sqrt(v + 1e-5) * g_ref[...] + be_ref[...]
    t = jnp.maximum(t, 0.0)
    ho = jnp.dot(t, W2_ref[...], preferred_element_type=jnp.float32)
    ho = ho + b2_ref[...]
    hout_ref[...] = ho
    onehot = (lax.broadcasted_iota(jnp.int32, (G, N), 0)
              == batch_ref[...]).astype(jnp.float32)
    pooled_ref[...] = jnp.dot(onehot, ho, preferred_element_type=jnp.float32, precision=lax.Precision.HIGHEST)


def _tc_layer(h, aggr2, batch_row, W1, b1, g, be, W2, b2):
    return pl.pallas_call(
        _tc_layer_body,
        out_shape=(
            jax.ShapeDtypeStruct((N, D), jnp.float32),
            jax.ShapeDtypeStruct((G, D), jnp.float32),
        ),
    )(h, aggr2, batch_row, W1, b1.reshape(1, D), g.reshape(1, D),
      be.reshape(1, D), W2, b2.reshape(1, D))


def _tc_final_body(xc_ref, Wp1_ref, bp1_ref, gp_ref, bep_ref, Wp2_ref,
                   bp2_ref, out_ref):
    t = jnp.dot(xc_ref[...], Wp1_ref[...], preferred_element_type=jnp.float32)
    t = t + bp1_ref[...]
    m = jnp.mean(t, axis=0, keepdims=True)
    v = jnp.mean((t - m) * (t - m), axis=0, keepdims=True)
    t = (t - m) / jnp.sqrt(v + 1e-5) * gp_ref[...] + bep_ref[...]
    t = jnp.maximum(t, 0.0)
    o = jnp.dot(t, Wp2_ref[...], preferred_element_type=jnp.float32)
    out_ref[...] = o + bp2_ref[...]


def _tc_final(xc, Wp1, bp1, gp, bep, Wp2, bp2):
    OUT = Wp2.shape[1]
    return pl.pallas_call(
        _tc_final_body,
        out_shape=jax.ShapeDtypeStruct((G, OUT), jnp.float32),
    )(xc, Wp1, bp1.reshape(1, -1), gp.reshape(1, -1), bep.reshape(1, -1),
      Wp2, bp2.reshape(1, -1))


def kernel(x, edge_index, batch, W1_0, b1_0, g_0, be_0, W2_0, b2_0,
           W1_1, b1_1, g_1, be_1, W2_1, b2_1,
           W1_2, b1_2, g_2, be_2, W2_2, b2_2,
           Wp1, bp1, gp, bep, Wp2, bp2):
    pad = EPAD - E
    src = jnp.concatenate([edge_index[0], jnp.zeros((pad,), jnp.int32)])
    dst = jnp.concatenate([edge_index[1], jnp.full((pad,), DUMMY, jnp.int32)])
    zeros = jnp.zeros((NPAD, D), jnp.float32)
    batch_row = batch.reshape(1, N)

    params = [
        (W1_0, b1_0, g_0, be_0, W2_0, b2_0),
        (W1_1, b1_1, g_1, be_1, W2_1, b2_1),
        (W1_2, b1_2, g_2, be_2, W2_2, b2_2),
    ]
    h = x
    pooled = []
    for (W1, b1, g, be, W2, b2) in params:
        aggr2 = _sc_aggregate(h, src, dst, zeros)
        h, p = _tc_layer(h, aggr2, batch_row, W1, b1, g, be, W2, b2)
        pooled.append(p)

    xc = jnp.concatenate(pooled, axis=1)
    return _tc_final(xc, Wp1, bp1, gp, bep, Wp2, bp2)


# speedup vs baseline: 2.0512x; 2.0512x over previous
import functools

import jax
import jax.numpy as jnp
from jax import lax
from jax.experimental import pallas as pl
from jax.experimental.pallas import tpu as pltpu
from jax.experimental.pallas import tpu_sc as plsc

N = 10000
E = 320000
D = 128
G = 64

NC = 2
NS = 16
NW = NC * NS
K = 80
NCHUNK = 125
EPW = NCHUNK * K
EPAD = NW * EPW
NPAD = 10240
RPT = NPAD // NS
DUMMY = NPAD - 8


def _sc_aggregate(h, src, dst, zeros):
    mesh = plsc.VectorSubcoreMesh(core_axis_name="c", subcore_axis_name="s")

    @functools.partial(
        pl.kernel,
        out_type=jax.ShapeDtypeStruct((NC, NPAD, D), jnp.float32),
        mesh=mesh,
        scratch_types=[
            pltpu.VMEM((K,), jnp.int32),
            pltpu.VMEM((K,), jnp.int32),
            pltpu.VMEM((K, D), jnp.float32),
            pltpu.VMEM_SHARED((NPAD, D), jnp.float32),
            pltpu.SemaphoreType.DMA,
        ],
    )
    def agg(h_hbm, src_hbm, dst_hbm, zero_hbm, out_hbm,
            src_v, dst_v, rows_v, acc_sh, sem):
        c = lax.axis_index("c")
        s = lax.axis_index("s")
        wid = s * NC + c
        r0 = s * RPT
        pltpu.sync_copy(zero_hbm.at[pl.ds(r0, RPT)], acc_sh.at[pl.ds(r0, RPT)])
        plsc.subcore_barrier()

        base0 = wid * EPW

        @pl.loop(0, NCHUNK)
        def chunk(i):
            base = base0 + i * K
            pltpu.sync_copy(src_hbm.at[pl.ds(base, K)], src_v)
            pltpu.sync_copy(dst_hbm.at[pl.ds(base, K)], dst_v)
            pltpu.async_copy(h_hbm.at[src_v], rows_v, sem).wait()
            pltpu.sync_copy(rows_v, acc_sh.at[dst_v], add=True)

        plsc.subcore_barrier()
        pltpu.sync_copy(acc_sh.at[pl.ds(r0, RPT)],
                        out_hbm.at[c, pl.ds(r0, RPT)])

    return agg(h, src, dst, zeros)


def _tc_layer_body(h_ref, a_ref, batch_ref, W1_ref, b1_ref, g_ref, be_ref,
                   W2_ref, b2_ref, hout_ref, pooled_ref):
    z = h_ref[...] + a_ref[0, :N] + a_ref[1, :N]
    t = jnp.dot(z, W1_ref[...], preferred_element_type=jnp.float32)
    t = t + b1_ref[...]
    m = jnp.mean(t, axis=0, keepdims=True)
    v = jnp.mean((t - m) * (t - m), axis=0, keepdims=True)
    t = (t - m) / jnp.sqrt(v + 1e-5) * g_ref[...] + be_ref[...]
    t = jnp.maximum(t, 0.0)
    ho = jnp.dot(t, W2_ref[...], preferred_element_type=jnp.float32)
    ho = ho + b2_ref[...]
    hout_ref[...] = ho
    onehot = (lax.broadcasted_iota(jnp.int32, (G, N), 0)
              == batch_ref[...]).astype(jnp.float32)
    pooled_ref[...] = jnp.dot(onehot, ho, preferred_element_type=jnp.float32, precision=lax.Precision.HIGHEST)


def _tc_layer(h, aggr2, batch_row, W1, b1, g, be, W2, b2):
    return pl.pallas_call(
        _tc_layer_body,
        out_shape=(
            jax.ShapeDtypeStruct((N, D), jnp.float32),
            jax.ShapeDtypeStruct((G, D), jnp.float32),
        ),
    )(h, aggr2, batch_row, W1, b1.reshape(1, D), g.reshape(1, D),
      be.reshape(1, D), W2, b2.reshape(1, D))


def _tc_final_body(xc_ref, Wp1_ref, bp1_ref, gp_ref, bep_ref, Wp2_ref,
                   bp2_ref, out_ref):
    t = jnp.dot(xc_ref[...], Wp1_ref[...], preferred_element_type=jnp.float32)
    t = t + bp1_ref[...]
    m = jnp.mean(t, axis=0, keepdims=True)
    v = jnp.mean((t - m) * (t - m), axis=0, keepdims=True)
    t = (t - m) / jnp.sqrt(v + 1e-5) * gp_ref[...] + bep_ref[...]
    t = jnp.maximum(t, 0.0)
    o = jnp.dot(t, Wp2_ref[...], preferred_element_type=jnp.float32)
    out_ref[...] = o + bp2_ref[...]


def _tc_final(xc, Wp1, bp1, gp, bep, Wp2, bp2):
    OUT = Wp2.shape[1]
    return pl.pallas_call(
        _tc_final_body,
        out_shape=jax.ShapeDtypeStruct((G, OUT), jnp.float32),
    )(xc, Wp1, bp1.reshape(1, -1), gp.reshape(1, -1), bep.reshape(1, -1),
      Wp2, bp2.reshape(1, -1))


def kernel(x, edge_index, batch, W1_0, b1_0, g_0, be_0, W2_0, b2_0,
           W1_1, b1_1, g_1, be_1, W2_1, b2_1,
           W1_2, b1_2, g_2, be_2, W2_2, b2_2,
           Wp1, bp1, gp, bep, Wp2, bp2):
    pad = EPAD - E
    src = jnp.concatenate([edge_index[0], jnp.zeros((pad,), jnp.int32)])
    dst = jnp.concatenate([edge_index[1], jnp.full((pad,), DUMMY, jnp.int32)])
    zeros = jnp.zeros((NPAD, D), jnp.float32)
    batch_row = batch.reshape(1, N)

    params = [
        (W1_0, b1_0, g_0, be_0, W2_0, b2_0),
        (W1_1, b1_1, g_1, be_1, W2_1, b2_1),
        (W1_2, b1_2, g_2, be_2, W2_2, b2_2),
    ]
    h = x
    pooled = []
    for (W1, b1, g, be, W2, b2) in params:
        aggr2 = _sc_aggregate(h, src, dst, zeros)
        h, p = _tc_layer(h, aggr2, batch_row, W1, b1, g, be, W2, b2)
        pooled.append(p)

    xc = jnp.concatenate(pooled, axis=1)
    return _tc_final(xc, Wp1, bp1, gp, bep, Wp2, bp2)


# speedup vs baseline: 2.3688x; 1.1549x over previous
import functools

import jax
import jax.numpy as jnp
from jax import lax
from jax.experimental import pallas as pl
from jax.experimental.pallas import tpu as pltpu
from jax.experimental.pallas import tpu_sc as plsc

N = 10000
E = 320000
D = 128
G = 64

NC = 2
NS = 16
NW = NC * NS
K = 112
NCHUNK = 92
EPW = NCHUNK * K
EPAD = NW * EPW
NPAD = 10240
RPT = NPAD // NS
DUMMY = NPAD - 8


def _sc_aggregate(h, src, dst, zeros):
    mesh = plsc.VectorSubcoreMesh(core_axis_name="c", subcore_axis_name="s")

    @functools.partial(
        pl.kernel,
        out_type=jax.ShapeDtypeStruct((NC, NPAD, D), jnp.float32),
        mesh=mesh,
        scratch_types=[
            pltpu.VMEM((K,), jnp.int32),
            pltpu.VMEM((K,), jnp.int32),
            pltpu.VMEM((K, D), jnp.float32),
            pltpu.VMEM_SHARED((NPAD, D), jnp.float32),
            pltpu.SemaphoreType.DMA,
        ],
    )
    def agg(h_hbm, src_hbm, dst_hbm, zero_hbm, out_hbm,
            src_v, dst_v, rows_v, acc_sh, sem):
        c = lax.axis_index("c")
        s = lax.axis_index("s")
        wid = s * NC + c
        r0 = s * RPT
        pltpu.sync_copy(zero_hbm.at[pl.ds(r0, RPT)], acc_sh.at[pl.ds(r0, RPT)])
        plsc.subcore_barrier()

        base0 = wid * EPW

        @pl.loop(0, NCHUNK)
        def chunk(i):
            base = base0 + i * K
            pltpu.sync_copy(src_hbm.at[pl.ds(base, K)], src_v)
            pltpu.sync_copy(dst_hbm.at[pl.ds(base, K)], dst_v)
            pltpu.async_copy(h_hbm.at[src_v], rows_v, sem).wait()
            pltpu.sync_copy(rows_v, acc_sh.at[dst_v], add=True)

        plsc.subcore_barrier()
        pltpu.sync_copy(acc_sh.at[pl.ds(r0, RPT)],
                        out_hbm.at[c, pl.ds(r0, RPT)])

    return agg(h, src, dst, zeros)


def _tc_layer_body(h_ref, a_ref, batch_ref, W1_ref, b1_ref, g_ref, be_ref,
                   W2_ref, b2_ref, hout_ref, pooled_ref):
    z = h_ref[...] + a_ref[0, :N] + a_ref[1, :N]
    t = jnp.dot(z, W1_ref[...], preferred_element_type=jnp.float32)
    t = t + b1_ref[...]
    m = jnp.mean(t, axis=0, keepdims=True)
    v = jnp.mean((t - m) * (t - m), axis=0, keepdims=True)
    t = (t - m) / jnp.sqrt(v + 1e-5) * g_ref[...] + be_ref[...]
    t = jnp.maximum(t, 0.0)
    ho = jnp.dot(t, W2_ref[...], preferred_element_type=jnp.float32)
    ho = ho + b2_ref[...]
    hout_ref[...] = ho
    onehot = (lax.broadcasted_iota(jnp.int32, (G, N), 0)
              == batch_ref[...]).astype(jnp.float32)
    pooled_ref[...] = jnp.dot(onehot, ho, preferred_element_type=jnp.float32, precision=lax.Precision.HIGHEST)


def _tc_layer(h, aggr2, batch_row, W1, b1, g, be, W2, b2):
    return pl.pallas_call(
        _tc_layer_body,
        out_shape=(
            jax.ShapeDtypeStruct((N, D), jnp.float32),
            jax.ShapeDtypeStruct((G, D), jnp.float32),
        ),
    )(h, aggr2, batch_row, W1, b1.reshape(1, D), g.reshape(1, D),
      be.reshape(1, D), W2, b2.reshape(1, D))


def _tc_final_body(xc_ref, Wp1_ref, bp1_ref, gp_ref, bep_ref, Wp2_ref,
                   bp2_ref, out_ref):
    t = jnp.dot(xc_ref[...], Wp1_ref[...], preferred_element_type=jnp.float32)
    t = t + bp1_ref[...]
    m = jnp.mean(t, axis=0, keepdims=True)
    v = jnp.mean((t - m) * (t - m), axis=0, keepdims=True)
    t = (t - m) / jnp.sqrt(v + 1e-5) * gp_ref[...] + bep_ref[...]
    t = jnp.maximum(t, 0.0)
    o = jnp.dot(t, Wp2_ref[...], preferred_element_type=jnp.float32)
    out_ref[...] = o + bp2_ref[...]


def _tc_final(xc, Wp1, bp1, gp, bep, Wp2, bp2):
    OUT = Wp2.shape[1]
    return pl.pallas_call(
        _tc_final_body,
        out_shape=jax.ShapeDtypeStruct((G, OUT), jnp.float32),
    )(xc, Wp1, bp1.reshape(1, -1), gp.reshape(1, -1), bep.reshape(1, -1),
      Wp2, bp2.reshape(1, -1))


def kernel(x, edge_index, batch, W1_0, b1_0, g_0, be_0, W2_0, b2_0,
           W1_1, b1_1, g_1, be_1, W2_1, b2_1,
           W1_2, b1_2, g_2, be_2, W2_2, b2_2,
           Wp1, bp1, gp, bep, Wp2, bp2):
    pad = EPAD - E
    src = jnp.concatenate(
        [edge_index[0], jnp.arange(pad, dtype=jnp.int32) % N])
    dst = jnp.concatenate(
        [edge_index[1], N + (jnp.arange(pad, dtype=jnp.int32) % (NPAD - N))])
    zeros = jnp.zeros((NPAD, D), jnp.float32)
    batch_row = batch.reshape(1, N)

    params = [
        (W1_0, b1_0, g_0, be_0, W2_0, b2_0),
        (W1_1, b1_1, g_1, be_1, W2_1, b2_1),
        (W1_2, b1_2, g_2, be_2, W2_2, b2_2),
    ]
    h = x
    pooled = []
    for (W1, b1, g, be, W2, b2) in params:
        aggr2 = _sc_aggregate(h, src, dst, zeros)
        h, p = _tc_layer(h, aggr2, batch_row, W1, b1, g, be, W2, b2)
        pooled.append(p)

    xc = jnp.concatenate(pooled, axis=1)
    return _tc_final(xc, Wp1, bp1, gp, bep, Wp2, bp2)


# speedup vs baseline: 2.4973x; 1.0542x over previous
import functools

import jax
import jax.numpy as jnp
from jax import lax
from jax.experimental import pallas as pl
from jax.experimental.pallas import tpu as pltpu
from jax.experimental.pallas import tpu_sc as plsc

N = 10000
E = 320000
D = 128
G = 64

NC = 2
NS = 16
NW = NC * NS
K = 120
NCHUNK = 84
EPW = NCHUNK * K
EPAD = NW * EPW
NPAD = 10240
RPT = NPAD // NS
DUMMY = NPAD - 8


def _sc_aggregate(h, src, dst, zeros):
    mesh = plsc.VectorSubcoreMesh(core_axis_name="c", subcore_axis_name="s")

    @functools.partial(
        pl.kernel,
        out_type=jax.ShapeDtypeStruct((NC, NPAD, D), jnp.float32),
        mesh=mesh,
        scratch_types=[
            pltpu.VMEM((K,), jnp.int32),
            pltpu.VMEM((K,), jnp.int32),
            pltpu.VMEM((K, D), jnp.float32),
            pltpu.VMEM_SHARED((NPAD, D), jnp.float32),
            pltpu.SemaphoreType.DMA,
        ],
    )
    def agg(h_hbm, src_hbm, dst_hbm, zero_hbm, out_hbm,
            src_v, dst_v, rows_v, acc_sh, sem):
        c = lax.axis_index("c")
        s = lax.axis_index("s")
        wid = s * NC + c
        r0 = s * RPT
        pltpu.sync_copy(zero_hbm.at[pl.ds(r0, RPT)], acc_sh.at[pl.ds(r0, RPT)])
        plsc.subcore_barrier()

        base0 = wid * EPW

        @pl.loop(0, NCHUNK)
        def chunk(i):
            base = base0 + i * K
            pltpu.sync_copy(src_hbm.at[pl.ds(base, K)], src_v)
            pltpu.sync_copy(dst_hbm.at[pl.ds(base, K)], dst_v)
            pltpu.async_copy(h_hbm.at[src_v], rows_v, sem).wait()
            pltpu.sync_copy(rows_v, acc_sh.at[dst_v], add=True)

        plsc.subcore_barrier()
        pltpu.sync_copy(acc_sh.at[pl.ds(r0, RPT)],
                        out_hbm.at[c, pl.ds(r0, RPT)])

    return agg(h, src, dst, zeros)


def _tc_layer_body(h_ref, a_ref, batch_ref, W1_ref, b1_ref, g_ref, be_ref,
                   W2_ref, b2_ref, hout_ref, pooled_ref):
    z = h_ref[...] + a_ref[0, :N] + a_ref[1, :N]
    t = jnp.dot(z, W1_ref[...], preferred_element_type=jnp.float32)
    t = t + b1_ref[...]
    m = jnp.mean(t, axis=0, keepdims=True)
    v = jnp.mean((t - m) * (t - m), axis=0, keepdims=True)
    t = (t - m) / jnp.sqrt(v + 1e-5) * g_ref[...] + be_ref[...]
    t = jnp.maximum(t, 0.0)
    ho = jnp.dot(t, W2_ref[...], preferred_element_type=jnp.float32)
    ho = ho + b2_ref[...]
    hout_ref[...] = ho
    onehot = (lax.broadcasted_iota(jnp.int32, (G, N), 0)
              == batch_ref[...]).astype(jnp.float32)
    pooled_ref[...] = jnp.dot(onehot, ho, preferred_element_type=jnp.float32, precision=lax.Precision.HIGHEST)


def _tc_layer(h, aggr2, batch_row, W1, b1, g, be, W2, b2):
    return pl.pallas_call(
        _tc_layer_body,
        out_shape=(
            jax.ShapeDtypeStruct((N, D), jnp.float32),
            jax.ShapeDtypeStruct((G, D), jnp.float32),
        ),
    )(h, aggr2, batch_row, W1, b1.reshape(1, D), g.reshape(1, D),
      be.reshape(1, D), W2, b2.reshape(1, D))


def _tc_final_body(xc_ref, Wp1_ref, bp1_ref, gp_ref, bep_ref, Wp2_ref,
                   bp2_ref, out_ref):
    t = jnp.dot(xc_ref[...], Wp1_ref[...], preferred_element_type=jnp.float32)
    t = t + bp1_ref[...]
    m = jnp.mean(t, axis=0, keepdims=True)
    v = jnp.mean((t - m) * (t - m), axis=0, keepdims=True)
    t = (t - m) / jnp.sqrt(v + 1e-5) * gp_ref[...] + bep_ref[...]
    t = jnp.maximum(t, 0.0)
    o = jnp.dot(t, Wp2_ref[...], preferred_element_type=jnp.float32)
    out_ref[...] = o + bp2_ref[...]


def _tc_final(xc, Wp1, bp1, gp, bep, Wp2, bp2):
    OUT = Wp2.shape[1]
    return pl.pallas_call(
        _tc_final_body,
        out_shape=jax.ShapeDtypeStruct((G, OUT), jnp.float32),
    )(xc, Wp1, bp1.reshape(1, -1), gp.reshape(1, -1), bep.reshape(1, -1),
      Wp2, bp2.reshape(1, -1))


def kernel(x, edge_index, batch, W1_0, b1_0, g_0, be_0, W2_0, b2_0,
           W1_1, b1_1, g_1, be_1, W2_1, b2_1,
           W1_2, b1_2, g_2, be_2, W2_2, b2_2,
           Wp1, bp1, gp, bep, Wp2, bp2):
    pad = EPAD - E
    src = jnp.concatenate(
        [edge_index[0], jnp.arange(pad, dtype=jnp.int32) % N])
    dst = jnp.concatenate(
        [edge_index[1], N + (jnp.arange(pad, dtype=jnp.int32) % (NPAD - N))])
    zeros = jnp.zeros((NPAD, D), jnp.float32)
    batch_row = batch.reshape(1, N)

    params = [
        (W1_0, b1_0, g_0, be_0, W2_0, b2_0),
        (W1_1, b1_1, g_1, be_1, W2_1, b2_1),
        (W1_2, b1_2, g_2, be_2, W2_2, b2_2),
    ]
    h = x
    pooled = []
    for (W1, b1, g, be, W2, b2) in params:
        aggr2 = _sc_aggregate(h, src, dst, zeros)
        h, p = _tc_layer(h, aggr2, batch_row, W1, b1, g, be, W2, b2)
        pooled.append(p)

    xc = jnp.concatenate(pooled, axis=1)
    return _tc_final(xc, Wp1, bp1, gp, bep, Wp2, bp2)


# speedup vs baseline: 3.8823x; 1.5546x over previous
import functools

import jax
import jax.numpy as jnp
from jax import lax
from jax.experimental import pallas as pl
from jax.experimental.pallas import tpu as pltpu
from jax.experimental.pallas import tpu_sc as plsc

N = 10000
E = 320000
D = 128
G = 64

NC = 2
NS = 16
NW = NC * NS
K = 120
NCHUNK = 84
EPW = NCHUNK * K
EPAD = NW * EPW
NPAD = 10240
RPT = NPAD // NS
DUMMY = NPAD - 8


def _sc_aggregate(h, src, dst, zeros):
    mesh = plsc.VectorSubcoreMesh(core_axis_name="c", subcore_axis_name="s")

    @functools.partial(
        pl.kernel,
        out_type=jax.ShapeDtypeStruct((NC, NPAD, D), jnp.float32),
        mesh=mesh,
        scratch_types=[
            [pltpu.VMEM((K,), jnp.int32)] * 2,
            [pltpu.VMEM((K,), jnp.int32)] * 2,
            [pltpu.VMEM((K, D), jnp.float32)] * 2,
            pltpu.VMEM_SHARED((NPAD, D), jnp.float32),
            [pltpu.SemaphoreType.DMA] * 2,
        ],
    )
    def agg(h_hbm, src_hbm, dst_hbm, zero_hbm, out_hbm,
            src_v, dst_v, rows_v, acc_sh, sems):
        c = lax.axis_index("c")
        s = lax.axis_index("s")
        wid = s * NC + c
        r0 = s * RPT
        pltpu.sync_copy(zero_hbm.at[pl.ds(r0, RPT)], acc_sh.at[pl.ds(r0, RPT)])
        plsc.subcore_barrier()

        base0 = wid * EPW

        for b in range(2):
            pltpu.sync_copy(src_hbm.at[pl.ds(base0 + b * K, K)], src_v[b])
            pltpu.sync_copy(dst_hbm.at[pl.ds(base0 + b * K, K)], dst_v[b])
            pltpu.async_copy(h_hbm.at[src_v[b]], rows_v[b], sems[b])

        @pl.loop(0, NCHUNK, step=2)
        def chunk(i):
            for b in range(2):
                pltpu.make_async_copy(h_hbm.at[src_v[b]], rows_v[b],
                                      sems[b]).wait()
                pltpu.sync_copy(rows_v[b], acc_sh.at[dst_v[b]], add=True)
                nxt = i + b + 2

                @pl.when(nxt < NCHUNK)
                def _():
                    base = base0 + nxt * K
                    pltpu.sync_copy(src_hbm.at[pl.ds(base, K)], src_v[b])
                    pltpu.sync_copy(dst_hbm.at[pl.ds(base, K)], dst_v[b])
                    pltpu.async_copy(h_hbm.at[src_v[b]], rows_v[b], sems[b])

        plsc.subcore_barrier()
        pltpu.sync_copy(acc_sh.at[pl.ds(r0, RPT)],
                        out_hbm.at[c, pl.ds(r0, RPT)])

    return agg(h, src, dst, zeros)


def _tc_layer_body(h_ref, a_ref, batch_ref, W1_ref, b1_ref, g_ref, be_ref,
                   W2_ref, b2_ref, hout_ref, pooled_ref):
    z = h_ref[...] + a_ref[0, :N] + a_ref[1, :N]
    t = jnp.dot(z, W1_ref[...], preferred_element_type=jnp.float32)
    t = t + b1_ref[...]
    m = jnp.mean(t, axis=0, keepdims=True)
    v = jnp.mean((t - m) * (t - m), axis=0, keepdims=True)
    t = (t - m) / jnp.sqrt(v + 1e-5) * g_ref[...] + be_ref[...]
    t = jnp.maximum(t, 0.0)
    ho = jnp.dot(t, W2_ref[...], preferred_element_type=jnp.float32)
    ho = ho + b2_ref[...]
    hout_ref[...] = ho
    onehot = (lax.broadcasted_iota(jnp.int32, (G, N), 0)
              == batch_ref[...]).astype(jnp.float32)
    pooled_ref[...] = jnp.dot(onehot, ho, preferred_element_type=jnp.float32, precision=lax.Precision.HIGHEST)


def _tc_layer(h, aggr2, batch_row, W1, b1, g, be, W2, b2):
    return pl.pallas_call(
        _tc_layer_body,
        out_shape=(
            jax.ShapeDtypeStruct((N, D), jnp.float32),
            jax.ShapeDtypeStruct((G, D), jnp.float32),
        ),
    )(h, aggr2, batch_row, W1, b1.reshape(1, D), g.reshape(1, D),
      be.reshape(1, D), W2, b2.reshape(1, D))


def _tc_final_body(xc_ref, Wp1_ref, bp1_ref, gp_ref, bep_ref, Wp2_ref,
                   bp2_ref, out_ref):
    t = jnp.dot(xc_ref[...], Wp1_ref[...], preferred_element_type=jnp.float32)
    t = t + bp1_ref[...]
    m = jnp.mean(t, axis=0, keepdims=True)
    v = jnp.mean((t - m) * (t - m), axis=0, keepdims=True)
    t = (t - m) / jnp.sqrt(v + 1e-5) * gp_ref[...] + bep_ref[...]
    t = jnp.maximum(t, 0.0)
    o = jnp.dot(t, Wp2_ref[...], preferred_element_type=jnp.float32)
    out_ref[...] = o + bp2_ref[...]


def _tc_final(xc, Wp1, bp1, gp, bep, Wp2, bp2):
    OUT = Wp2.shape[1]
    return pl.pallas_call(
        _tc_final_body,
        out_shape=jax.ShapeDtypeStruct((G, OUT), jnp.float32),
    )(xc, Wp1, bp1.reshape(1, -1), gp.reshape(1, -1), bep.reshape(1, -1),
      Wp2, bp2.reshape(1, -1))


def kernel(x, edge_index, batch, W1_0, b1_0, g_0, be_0, W2_0, b2_0,
           W1_1, b1_1, g_1, be_1, W2_1, b2_1,
           W1_2, b1_2, g_2, be_2, W2_2, b2_2,
           Wp1, bp1, gp, bep, Wp2, bp2):
    pad = EPAD - E
    src = jnp.concatenate(
        [edge_index[0], jnp.arange(pad, dtype=jnp.int32) % N])
    dst = jnp.concatenate(
        [edge_index[1], N + (jnp.arange(pad, dtype=jnp.int32) % (NPAD - N))])
    zeros = jnp.zeros((NPAD, D), jnp.float32)
    batch_row = batch.reshape(1, N)

    params = [
        (W1_0, b1_0, g_0, be_0, W2_0, b2_0),
        (W1_1, b1_1, g_1, be_1, W2_1, b2_1),
        (W1_2, b1_2, g_2, be_2, W2_2, b2_2),
    ]
    h = x
    pooled = []
    for (W1, b1, g, be, W2, b2) in params:
        aggr2 = _sc_aggregate(h, src, dst, zeros)
        h, p = _tc_layer(h, aggr2, batch_row, W1, b1, g, be, W2, b2)
        pooled.append(p)

    xc = jnp.concatenate(pooled, axis=1)
    return _tc_final(xc, Wp1, bp1, gp, bep, Wp2, bp2)
